# baseline (speedup 1.0000x reference)
import functools

import jax
import jax.numpy as jnp
from jax.experimental import pallas as pl
from jax.experimental.pallas import tpu as pltpu


_VMEM_LIMIT_BYTES = 48 * 1024 * 1024


def _cdiv(a, b):
    return (a + b - 1) // b


def _divisor_tile(n, cap):
    t = min(int(cap), max(int(n), 8)) // 8 * 8
    while t >= 8:
        if n % t == 0:
            return t
        t -= 8
    return 0


def _pool_kernel(vf_ref, bidx_ref, sum_ref, cnt_ref, *, n_rows, row_tile,
                 n_batches, ragged):
    tile = pl.program_id(0)
    vf = vf_ref[...]
    if ragged:
        row = tile * row_tile + jax.lax.broadcasted_iota(
            jnp.int32, (row_tile, 1), 0)
        vf = jnp.where(row < n_rows, vf, 0.0)

    bidx = bidx_ref[...].reshape(1, row_tile)
    onehot = jnp.where(
        bidx == jax.lax.broadcasted_iota(
            jnp.int32, (n_batches, row_tile), 0), 1.0, 0.0)

    sum_ref[...] = jax.lax.dot_general(
        onehot, vf, (((1,), (0,)), ((), ())),
        preferred_element_type=jnp.float32)[None]
    cnt_ref[...] = jnp.sum(onehot, axis=1, keepdims=True)[None]


def _scale_kernel(vf_ref, bidx_ref, scal_ref, out_ref, *, row_tile, n_batches):
    bidx = bidx_ref[...].reshape(1, row_tile)
    onehot = jnp.where(
        bidx == jax.lax.broadcasted_iota(
            jnp.int32, (n_batches, row_tile), 0), 1.0, 0.0)
    scale = jax.lax.dot_general(
        onehot, scal_ref[...], (((0,), (0,)), ((), ())),
        preferred_element_type=jnp.float32)
    out_ref[...] = vf_ref[...] * scale


def kernel(voxel_feat, batch_idx, global_feat, global_weight,
           w_glob, b_glob, w1, b1, w2, b2):
    n_voxels, channels = voxel_feat.shape
    n_batches = global_feat.shape[0]
    f32 = jnp.float32
    hi = jax.lax.Precision.HIGHEST
    batch_idx = batch_idx.astype(jnp.int32)

    def _tiling(cap):
        t = _divisor_tile(n_voxels, cap)
        ragged = t == 0
        if ragged:
            t = min(cap // 8 * 8, _cdiv(n_voxels, 8) * 8)
        n = _cdiv(n_voxels, t)
        pad = n * t - n_voxels
        lane = batch_idx if pad == 0 else jnp.pad(
            batch_idx, (0, pad), constant_values=n_batches)
        return t, n, ragged, lane.reshape(n, 1, t)

    t1, n_tiles, ragged, bidx_lane = _tiling(25000)

    sums, cnts = pl.pallas_call(
        functools.partial(_pool_kernel, n_rows=n_voxels,
                          row_tile=t1, n_batches=n_batches, ragged=ragged),
        grid=(n_tiles,),
        in_specs=[
            pl.BlockSpec((t1, channels), lambda i: (i, 0)),
            pl.BlockSpec((1, 1, t1), lambda i: (i, 0, 0)),
        ],
        out_specs=[
            pl.BlockSpec((1, n_batches, channels), lambda i: (i, 0, 0)),
            pl.BlockSpec((1, n_batches, 1), lambda i: (i, 0, 0)),
        ],
        out_shape=[
            jax.ShapeDtypeStruct((n_tiles, n_batches, channels), f32),
            jax.ShapeDtypeStruct((n_tiles, n_batches, 1), f32),
        ],
        compiler_params=pltpu.CompilerParams(
            dimension_semantics=("parallel",),
            vmem_limit_bytes=_VMEM_LIMIT_BYTES),
    )(voxel_feat, bidx_lane)

    pooled = sums.sum(axis=0) / jnp.maximum(cnts.sum(axis=0), 1.0)
    gt = (jnp.dot(global_feat.astype(f32), w_glob.T.astype(f32), precision=hi)
          + b_glob.astype(f32))
    gt = jnp.asarray(global_weight, f32) * gt
    combined = jnp.concatenate([pooled, gt], axis=1)
    hidden = jax.nn.gelu(
        jnp.dot(combined, w1.T.astype(f32), precision=hi)
        + b1.astype(f32), approximate=False)
    scaling = jax.nn.sigmoid(
        jnp.dot(hidden, w2.T.astype(f32), precision=hi)
        + b2.astype(f32))

    t2, n_tiles2, _, bidx_lane2 = _tiling(20000)
    out = pl.pallas_call(
        functools.partial(_scale_kernel, row_tile=t2, n_batches=n_batches),
        grid=(n_tiles2,),
        in_specs=[
            pl.BlockSpec((t2, channels), lambda i: (i, 0)),
            pl.BlockSpec((1, 1, t2), lambda i: (i, 0, 0)),
            pl.BlockSpec((n_batches, channels), lambda i: (0, 0)),
        ],
        out_specs=pl.BlockSpec((t2, channels), lambda i: (i, 0)),
        out_shape=jax.ShapeDtypeStruct((n_voxels, channels),
                                       voxel_feat.dtype),
        compiler_params=pltpu.CompilerParams(
            dimension_semantics=("parallel",),
            vmem_limit_bytes=_VMEM_LIMIT_BYTES),
    )(voxel_feat, bidx_lane2, scaling)

    return out, scaling


# speedup vs baseline: 1.0092x; 1.0092x over previous
import functools

import jax
import jax.numpy as jnp
from jax.experimental import pallas as pl
from jax.experimental.pallas import tpu as pltpu


_VMEM_LIMIT_BYTES = 48 * 1024 * 1024


def _cdiv(a, b):
    return (a + b - 1) // b


def _divisor_tile(n, cap):
    t = min(int(cap), max(int(n), 8)) // 8 * 8
    while t >= 8:
        if n % t == 0:
            return t
        t -= 8
    return 0


def _pool_kernel(vf_ref, bidx_ref, sum_ref, cnt_ref, *, n_rows, row_tile,
                 n_batches, ragged):
    tile = pl.program_id(0)
    vf = vf_ref[...]
    if ragged:
        row = tile * row_tile + jax.lax.broadcasted_iota(
            jnp.int32, (row_tile, 1), 0)
        vf = jnp.where(row < n_rows, vf, 0.0)

    bidx = bidx_ref[...].reshape(1, row_tile)
    onehot = jnp.where(
        bidx == jax.lax.broadcasted_iota(
            jnp.int32, (n_batches, row_tile), 0), 1.0, 0.0)

    sum_ref[...] = jax.lax.dot_general(
        onehot, vf, (((1,), (0,)), ((), ())),
        preferred_element_type=jnp.float32)[None]
    cnt_ref[...] = jnp.sum(onehot, axis=1, keepdims=True)[None]


def _scale_kernel(vf_ref, bidx_ref, scal_ref, out_ref, *, row_tile, n_batches):
    bidx = bidx_ref[...].reshape(1, row_tile)
    onehot = jnp.where(
        bidx == jax.lax.broadcasted_iota(
            jnp.int32, (n_batches, row_tile), 0), 1.0, 0.0)
    scale = jax.lax.dot_general(
        onehot, scal_ref[...], (((0,), (0,)), ((), ())),
        preferred_element_type=jnp.float32)
    out_ref[...] = vf_ref[...] * scale


def kernel(voxel_feat, batch_idx, global_feat, global_weight,
           w_glob, b_glob, w1, b1, w2, b2):
    n_voxels, channels = voxel_feat.shape
    n_batches = global_feat.shape[0]
    f32 = jnp.float32
    hi = jax.lax.Precision.HIGHEST
    batch_idx = batch_idx.astype(jnp.int32)

    def _tiling(cap):
        t = _divisor_tile(n_voxels, cap)
        ragged = t == 0
        if ragged:
            t = min(cap // 8 * 8, _cdiv(n_voxels, 8) * 8)
        n = _cdiv(n_voxels, t)
        pad = n * t - n_voxels
        lane = batch_idx if pad == 0 else jnp.pad(
            batch_idx, (0, pad), constant_values=n_batches)
        return t, n, ragged, lane.reshape(n, 1, t)

    t1, n_tiles, ragged, bidx_lane = _tiling(20000)

    sums, cnts = pl.pallas_call(
        functools.partial(_pool_kernel, n_rows=n_voxels,
                          row_tile=t1, n_batches=n_batches, ragged=ragged),
        grid=(n_tiles,),
        in_specs=[
            pl.BlockSpec((t1, channels), lambda i: (i, 0)),
            pl.BlockSpec((1, 1, t1), lambda i: (i, 0, 0)),
        ],
        out_specs=[
            pl.BlockSpec((1, n_batches, channels), lambda i: (i, 0, 0)),
            pl.BlockSpec((1, n_batches, 1), lambda i: (i, 0, 0)),
        ],
        out_shape=[
            jax.ShapeDtypeStruct((n_tiles, n_batches, channels), f32),
            jax.ShapeDtypeStruct((n_tiles, n_batches, 1), f32),
        ],
        compiler_params=pltpu.CompilerParams(
            dimension_semantics=("parallel",),
            vmem_limit_bytes=_VMEM_LIMIT_BYTES),
    )(voxel_feat, bidx_lane)

    pooled = sums.sum(axis=0) / jnp.maximum(cnts.sum(axis=0), 1.0)
    gt = (jnp.dot(global_feat.astype(f32), w_glob.T.astype(f32), precision=hi)
          + b_glob.astype(f32))
    gt = jnp.asarray(global_weight, f32) * gt
    combined = jnp.concatenate([pooled, gt], axis=1)
    hidden = jax.nn.gelu(
        jnp.dot(combined, w1.T.astype(f32), precision=hi)
        + b1.astype(f32), approximate=False)
    scaling = jax.nn.sigmoid(
        jnp.dot(hidden, w2.T.astype(f32), precision=hi)
        + b2.astype(f32))

    out = pl.pallas_call(
        functools.partial(_scale_kernel, row_tile=t1, n_batches=n_batches),
        grid=(n_tiles,),
        in_specs=[
            pl.BlockSpec((t1, channels), lambda i: (i, 0)),
            pl.BlockSpec((1, 1, t1), lambda i: (i, 0, 0)),
            pl.BlockSpec((n_batches, channels), lambda i: (0, 0)),
        ],
        out_specs=pl.BlockSpec((t1, channels), lambda i: (i, 0)),
        out_shape=jax.ShapeDtypeStruct((n_voxels, channels),
                                       voxel_feat.dtype),
        compiler_params=pltpu.CompilerParams(
            dimension_semantics=("parallel",),
            vmem_limit_bytes=_VMEM_LIMIT_BYTES),
    )(voxel_feat, bidx_lane, scaling)

    return out, scaling
